# baseline (speedup 1.0000x reference)
import functools

import jax
import jax.numpy as jnp
from jax import lax
from jax.experimental import pallas as pl
from jax.experimental.pallas import tpu as pltpu
from jax.experimental.pallas import tpu_sc as plsc

_EMBED = 128

_MESH = plsc.ScalarSubcoreMesh(axis_name="c", num_cores=1)


@functools.partial(
    pl.kernel,
    out_type=jax.ShapeDtypeStruct((1, _EMBED), jnp.float32),
    mesh=_MESH,
    scratch_types=[
        pltpu.SMEM((1,), jnp.int32),
        pltpu.VMEM_SHARED((2, _EMBED), jnp.float32),
        pltpu.SemaphoreType.DMA,
        pltpu.SemaphoreType.DMA,
    ],
    compiler_params=pltpu.CompilerParams(skip_device_barrier=True),
)
def _lookup(idx_hbm, table_hbm, out_hbm, idx_s, tab_sp, sem_i, sem_t):
    cp_i = pltpu.async_copy(idx_hbm, idx_s, sem_i)
    cp_t = pltpu.async_copy(table_hbm, tab_sp, sem_t)
    cp_i.wait()
    cp_t.wait()
    i = idx_s[0]
    pltpu.sync_copy(tab_sp.at[pl.ds(i, 1)], out_hbm)


def kernel(mask_embeddings, is_present):
    idx = jnp.asarray(is_present, jnp.int32).reshape(1)
    return _lookup(idx, mask_embeddings).reshape(_EMBED)


# speedup vs baseline: 1.0017x; 1.0017x over previous
import functools

import jax
import jax.numpy as jnp
from jax import lax
from jax.experimental import pallas as pl
from jax.experimental.pallas import tpu as pltpu
from jax.experimental.pallas import tpu_sc as plsc

_EMBED = 128

_MESH = plsc.ScalarSubcoreMesh(axis_name="c", num_cores=1)


@functools.partial(
    pl.kernel,
    out_type=jax.ShapeDtypeStruct((1, _EMBED), jnp.float32),
    mesh=_MESH,
    scratch_types=[
        pltpu.SMEM((1,), jnp.int32),
        pltpu.SemaphoreType.DMA,
        pltpu.SemaphoreType.DMA,
    ],
    compiler_params=pltpu.CompilerParams(skip_device_barrier=True),
)
def _lookup(idx_hbm, table_hbm, out_hbm, idx_s, sem_i, sem_t):
    cp_i = pltpu.async_copy(idx_hbm, idx_s, sem_i)
    cp_t = pltpu.async_copy(table_hbm.at[pl.ds(1, 1)], out_hbm, sem_t)
    cp_i.wait()
    cp_t.wait()

    @pl.when(idx_s[0] == 0)
    def _():
        pltpu.sync_copy(table_hbm.at[pl.ds(0, 1)], out_hbm)


def kernel(mask_embeddings, is_present):
    idx = jnp.asarray(is_present, jnp.int32).reshape(1)
    return _lookup(idx, mask_embeddings).reshape(_EMBED)


# speedup vs baseline: 9.1813x; 9.1657x over previous
import jax
import jax.numpy as jnp
from jax.experimental import pallas as pl
from jax.experimental.pallas import tpu as pltpu

_EMBED = 128


def _tc_body(idx_ref, tab_ref, out_ref):
    i = idx_ref[0]
    out_ref[:, :] = jnp.where(i == 1, tab_ref[1:2, :], tab_ref[0:1, :])


def kernel(mask_embeddings, is_present):
    idx = jnp.asarray(is_present, jnp.int32).reshape(1)
    out = pl.pallas_call(
        _tc_body,
        out_shape=jax.ShapeDtypeStruct((1, _EMBED), jnp.float32),
        in_specs=[
            pl.BlockSpec(memory_space=pltpu.SMEM),
            pl.BlockSpec(memory_space=pltpu.ANY if False else pltpu.VMEM),
        ],
        out_specs=pl.BlockSpec(memory_space=pltpu.VMEM),
    )(idx, mask_embeddings)
    return out.reshape(_EMBED)
